# Initial kernel scaffold; baseline (speedup 1.0000x reference)
import jax
import jax.numpy as jnp
from jax.experimental import pallas as pl


def kernel(indices, W):
    raise NotImplementedError("write your pallas kernel here")


# speedup vs baseline: 2.9272x; 2.9272x over previous
import functools

import jax
import jax.numpy as jnp
from jax import lax
from jax.experimental import pallas as pl
from jax.experimental.pallas import tpu as pltpu
from jax.experimental.pallas import tpu_sc as plsc

BATCH = 16384
HIST = 50
EMBED = 32
LANES = 16
NCORES = 2
NSUB = 16
NWORKERS = NCORES * NSUB
ROWS_PER_CHUNK = 2
CHUNK = ROWS_PER_CHUNK * HIST
CHUNKS = 16
ROWS_PER_BLOCK = ROWS_PER_CHUNK * CHUNKS
BLOCKS_TOTAL = BATCH // ROWS_PER_BLOCK
BLOCKS_PER_W = BLOCKS_TOTAL // NWORKERS


def _sc_body(idx_hbm, w_hbm, out_hbm, idx_v, rows_v, outb_v, sg0, sg1, so0, so1):
    sg = (sg0, sg1)
    so = (so0, so1)
    wid = lax.axis_index("s") * NCORES + lax.axis_index("c")
    blk0 = wid * BLOCKS_PER_W

    def stage(blk, b):
        pltpu.sync_copy(idx_hbm.at[blk], idx_v.at[b])

        def fire(k, carry):
            pltpu.make_async_copy(
                w_hbm.at[idx_v.at[b, k]], rows_v.at[b, k], sg[b]).start()
            return carry

        lax.fori_loop(0, CHUNKS, fire, 0)

    def drain(b):
        def wait_one(k, carry):
            pltpu.make_async_copy(
                w_hbm.at[idx_v.at[b, k]], rows_v.at[b, k], sg[b]).wait()
            return carry

        lax.fori_loop(0, CHUNKS, wait_one, 0)

    def compute(b):
        def chunk_body(k, carry):
            for r in range(ROWS_PER_CHUNK):
                base = r * HIST
                a0 = rows_v[b, k, base, 0:LANES]
                a1 = rows_v[b, k, base, LANES:2 * LANES]
                for j in range(1, HIST):
                    a0 = a0 + rows_v[b, k, base + j, 0:LANES]
                    a1 = a1 + rows_v[b, k, base + j, LANES:2 * LANES]
                outb_v[b, ROWS_PER_CHUNK * k + r, 0:LANES] = a0
                outb_v[b, ROWS_PER_CHUNK * k + r, LANES:2 * LANES] = a1
            return carry

        lax.fori_loop(0, CHUNKS, chunk_body, 0)

    def out_copy(blk, b):
        return pltpu.make_async_copy(
            outb_v.at[b],
            out_hbm.at[pl.ds(blk * ROWS_PER_BLOCK, ROWS_PER_BLOCK)],
            so[b])

    stage(blk0, 0)

    def outer(g2, carry):
        for b in range(2):
            g = g2 * 2 + b
            blk = blk0 + g

            @pl.when(g < BLOCKS_PER_W - 1)
            def _():
                stage(blk + 1, 1 - b)

            drain(b)

            @pl.when(g >= 2)
            def _():
                out_copy(blk - 2, b).wait()

            compute(b)
            out_copy(blk, b).start()
        return carry

    lax.fori_loop(0, BLOCKS_PER_W // 2, outer, 0)
    for b in range(2):
        out_copy(blk0 + BLOCKS_PER_W - 2 + b, b).wait()


@jax.jit
def _embed_sum(idx3, W):
    mesh = plsc.VectorSubcoreMesh(
        core_axis_name="c", subcore_axis_name="s",
        num_cores=NCORES, num_subcores=NSUB)
    return pl.kernel(
        _sc_body,
        out_type=jax.ShapeDtypeStruct((BATCH, EMBED), jnp.float32),
        mesh=mesh,
        compiler_params=pltpu.CompilerParams(use_tc_tiling_on_sc=False),
        scratch_types=[
            pltpu.VMEM((2, CHUNKS, CHUNK), jnp.int32),
            pltpu.VMEM((2, CHUNKS, CHUNK, EMBED), jnp.float32),
            pltpu.VMEM((2, ROWS_PER_BLOCK, EMBED), jnp.float32),
            pltpu.SemaphoreType.DMA,
            pltpu.SemaphoreType.DMA,
            pltpu.SemaphoreType.DMA,
            pltpu.SemaphoreType.DMA,
        ],
    )(idx3, W)


def kernel(indices, W):
    idx3 = indices.astype(jnp.int32).reshape(BLOCKS_TOTAL, CHUNKS, CHUNK)
    return _embed_sum(idx3, W)
